# Initial kernel scaffold; baseline (speedup 1.0000x reference)
import jax
import jax.numpy as jnp
from jax.experimental import pallas as pl


def kernel(x1, edge_index1, edge_attr1, x2, edge_index2, edge_attr2, WA0, bA0, WA1, bA1, WB0, bB0, WB1, bB1, Wfc, bfc):
    raise NotImplementedError("write your pallas kernel here")


# speedup vs baseline: 4.7465x; 4.7465x over previous
import functools

import jax
import jax.numpy as jnp
from jax import lax
from jax.experimental import pallas as pl
from jax.experimental.pallas import tpu as pltpu
from jax.experimental.pallas import tpu_sc as plsc

N = 10000
D = 128
DH = 64
DE = 16
E = 320000

NC = 2
NS = 16
CHUNK = 128
NCH = 158
EPT = NCH * CHUNK
PAD = NS * EPT - E
NPAD = 10112
RPT = NPAD // NS
NCH_LAST = (E - (NS - 1) * EPT) // CHUNK


def _edge_pass(c, s, tabA, tabB, src_t, dst_t, rows, gsem, acc,
               ea_parts=None):
    if ea_parts is not None:
        eaA, eaB, eab, esem, accE = ea_parts
        nreal = jnp.where(s == NS - 1, NCH_LAST, NCH)

    def start(j, b):
        @pl.when(c == 0)
        def _():
            pltpu.async_copy(tabA.at[src_t.at[j]], rows[b], gsem[b])

        @pl.when(c != 0)
        def _():
            pltpu.async_copy(tabB.at[src_t.at[j]], rows[b], gsem[b])

        if ea_parts is not None:
            @pl.when(j < nreal)
            def _():
                off = s * EPT + j * CHUNK

                @pl.when(c == 0)
                def _():
                    pltpu.async_copy(eaA.at[pl.ds(off, CHUNK)], eab[b],
                                     esem[b])

                @pl.when(c != 0)
                def _():
                    pltpu.async_copy(eaB.at[pl.ds(off, CHUNK)], eab[b],
                                     esem[b])

    start(0, 0)
    start(1, 1)

    @pl.loop(0, NCH // 2)
    def _(i):
        for b in range(2):
            j = 2 * i + b
            pltpu.make_async_copy(tabA.at[src_t.at[j]], rows[b],
                                  gsem[b]).wait()
            pltpu.sync_copy(rows[b], acc.at[dst_t.at[j]], add=True)
            if ea_parts is not None:
                @pl.when(j < nreal)
                def _():
                    pltpu.make_async_copy(eaA.at[pl.ds(0, CHUNK)], eab[b],
                                          esem[b]).wait()
                    pltpu.sync_copy(eab[b], accE.at[dst_t.at[j]], add=True)

            @pl.when(j + 2 < NCH)
            def _():
                start(j + 2, b)


def _sc_body(include_ea, *refs):
    if include_ea:
        (tAlo, tAhi, tBlo, tBhi, src4, dst4, eaA, eaB, z64, z16,
         oXlo, oXhi, outE,
         src_t, dst_t, r0, r1, e0, e1, accX, accE, g0, g1, s0, s1) = refs
    else:
        (tAlo, tAhi, tBlo, tBhi, src4, dst4, z64,
         oXlo, oXhi,
         src_t, dst_t, r0, r1, accX, g0, g1) = refs

    c = lax.axis_index("c")
    s = lax.axis_index("s")
    rb = s * RPT
    rows = (r0, r1)
    gsem = (g0, g1)

    pltpu.sync_copy(src4.at[c, s], src_t)
    pltpu.sync_copy(dst4.at[c, s], dst_t)

    for p in range(2):
        tabA = tAlo if p == 0 else tAhi
        tabB = tBlo if p == 0 else tBhi
        outX = oXlo if p == 0 else oXhi
        ea_parts = None
        if include_ea and p == 0:
            ea_parts = (eaA, eaB, (e0, e1), (s0, s1), accE)

        pltpu.sync_copy(z64.at[pl.ds(rb, RPT)], accX.at[pl.ds(rb, RPT)])
        if include_ea and p == 0:
            pltpu.sync_copy(z16.at[pl.ds(rb, RPT)], accE.at[pl.ds(rb, RPT)])
        plsc.subcore_barrier()

        _edge_pass(c, s, tabA, tabB, src_t, dst_t, rows, gsem, accX,
                   ea_parts)

        plsc.subcore_barrier()
        pltpu.sync_copy(accX.at[pl.ds(rb, RPT)], outX.at[c, pl.ds(rb, RPT)])
        if include_ea and p == 0:
            pltpu.sync_copy(accE.at[pl.ds(rb, RPT)],
                            outE.at[c, pl.ds(rb, RPT)])


@functools.cache
def _sc_kernel(include_ea):
    mesh = plsc.VectorSubcoreMesh(
        core_axis_name="c", subcore_axis_name="s", num_cores=NC,
        num_subcores=NS)
    f32 = jnp.float32
    out_type = [jax.ShapeDtypeStruct((NC, NPAD, DH), f32),
                jax.ShapeDtypeStruct((NC, NPAD, DH), f32)]
    scratch = [
        pltpu.VMEM((NCH, CHUNK), jnp.int32),
        pltpu.VMEM((NCH, CHUNK), jnp.int32),
        pltpu.VMEM((CHUNK, DH), f32),
        pltpu.VMEM((CHUNK, DH), f32),
    ]
    if include_ea:
        out_type.append(jax.ShapeDtypeStruct((NC, NPAD, DE), f32))
        scratch += [pltpu.VMEM((CHUNK, DE), f32), pltpu.VMEM((CHUNK, DE), f32)]
    scratch.append(pltpu.VMEM_SHARED((NPAD, DH), f32))
    if include_ea:
        scratch.append(pltpu.VMEM_SHARED((NPAD, DE), f32))
    scratch += [pltpu.SemaphoreType.DMA, pltpu.SemaphoreType.DMA]
    if include_ea:
        scratch += [pltpu.SemaphoreType.DMA, pltpu.SemaphoreType.DMA]
    return pl.kernel(
        functools.partial(_sc_body, include_ea),
        out_type=tuple(out_type),
        mesh=mesh,
        scratch_types=scratch,
        compiler_params=pltpu.CompilerParams(use_tc_tiling_on_sc=False),
    )


def _tc1_body(axlo, axhi, ae, wxlo, wxhi, we, b, oalo, oahi, oblo, obhi):
    outs = ((oalo, oahi), (oblo, obhi))
    for g in range(2):
        y = jnp.dot(axlo[g], wxlo[g], preferred_element_type=jnp.float32)
        y = y + jnp.dot(axhi[g], wxhi[g], preferred_element_type=jnp.float32)
        y = y + jnp.dot(ae[g], we[g], preferred_element_type=jnp.float32)
        y = y + b[g]
        y = jnp.where(y >= 0, y, 0.01 * y)
        outs[g][0][...] = y[:, :DH]
        outs[g][1][...] = y[:, DH:]


def _tc2_body(axlo, axhi, ae, wxlo, wxhi, we, b, wfc, bfc, o1, o2):
    for g in range(2):
        y = jnp.dot(axlo[g], wxlo[g], preferred_element_type=jnp.float32)
        y = y + jnp.dot(axhi[g], wxhi[g], preferred_element_type=jnp.float32)
        y = y + jnp.dot(ae[g], we[g], preferred_element_type=jnp.float32)
        y = y + b[g]
        y = jnp.where(y >= 0, y, 0.01 * y)
        o = jnp.dot(y, wfc[...], preferred_element_type=jnp.float32) + bfc[...]
        (o1 if g == 0 else o2)[...] = o


_RB = 1000


def _tc_specs():
    return [
        pl.BlockSpec((NC, _RB, DH), lambda r: (0, r, 0)),
        pl.BlockSpec((NC, _RB, DH), lambda r: (0, r, 0)),
        pl.BlockSpec((NC, _RB, DE), lambda r: (0, r, 0)),
        pl.BlockSpec((NC, DH, D), lambda r: (0, 0, 0)),
        pl.BlockSpec((NC, DH, D), lambda r: (0, 0, 0)),
        pl.BlockSpec((NC, DE, D), lambda r: (0, 0, 0)),
        pl.BlockSpec((NC, 1, D), lambda r: (0, 0, 0)),
    ]


def _tc1(axlo, axhi, ae, wxlo, wxhi, we, b):
    f32 = jnp.float32
    half = pl.BlockSpec((_RB, DH), lambda r: (r, 0))
    return pl.pallas_call(
        _tc1_body,
        grid=(N // _RB,),
        in_specs=_tc_specs(),
        out_specs=(half, half, half, half),
        out_shape=tuple(jax.ShapeDtypeStruct((N, DH), f32) for _ in range(4)),
    )(axlo, axhi, ae, wxlo, wxhi, we, b)


def _tc2(axlo, axhi, ae, wxlo, wxhi, we, b, wfc, bfc):
    f32 = jnp.float32
    return pl.pallas_call(
        _tc2_body,
        grid=(N // _RB,),
        in_specs=_tc_specs() + [
            pl.BlockSpec((D, D), lambda r: (0, 0)),
            pl.BlockSpec((1, D), lambda r: (0, 0)),
        ],
        out_specs=(pl.BlockSpec((_RB, D), lambda r: (r, 0)),
                   pl.BlockSpec((_RB, D), lambda r: (r, 0))),
        out_shape=(jax.ShapeDtypeStruct((N, D), f32),
                   jax.ShapeDtypeStruct((N, D), f32)),
    )(axlo, axhi, ae, wxlo, wxhi, we, b, wfc, bfc)


def kernel(x1, edge_index1, edge_attr1, x2, edge_index2, edge_attr2,
           WA0, bA0, WA1, bA1, WB0, bB0, WB1, bB1, Wfc, bfc):
    f32 = jnp.float32
    i32 = jnp.int32
    src1 = edge_index1[0].astype(i32)
    dst1 = edge_index1[1].astype(i32)
    src2 = edge_index2[0].astype(i32)
    dst2 = edge_index2[1].astype(i32)
    pad_src = jnp.zeros((PAD,), i32)
    pad_dst = jnp.full((PAD,), N, i32)
    src4 = jnp.stack([jnp.concatenate([src1, pad_src]),
                      jnp.concatenate([src2, pad_src])]).reshape(
                          NC, NS, NCH, CHUNK)
    dst4 = jnp.stack([jnp.concatenate([dst1, pad_dst]),
                      jnp.concatenate([dst2, pad_dst])]).reshape(
                          NC, NS, NCH, CHUNK)
    z64 = jnp.zeros((NPAD, DH), f32)
    z16 = jnp.zeros((NPAD, DE), f32)

    wxlo0 = jnp.stack([WA0[:, :DH].T, WB0[:, :DH].T])
    wxhi0 = jnp.stack([WA0[:, DH:D].T, WB0[:, DH:D].T])
    we0 = jnp.stack([WA0[:, D:].T, WB0[:, D:].T])
    b0 = jnp.stack([bA0, bB0]).reshape(NC, 1, D)
    wxlo1 = jnp.stack([WA1[:, :DH].T, WB1[:, :DH].T])
    wxhi1 = jnp.stack([WA1[:, DH:D].T, WB1[:, DH:D].T])
    we1 = jnp.stack([WA1[:, D:].T, WB1[:, D:].T])
    b1 = jnp.stack([bA1, bB1]).reshape(NC, 1, D)
    wfcT = Wfc.T
    bfc2 = bfc.reshape(1, D)

    x1lo, x1hi = x1[:, :DH], x1[:, DH:]
    x2lo, x2hi = x2[:, :DH], x2[:, DH:]

    aXlo, aXhi, aE = _sc_kernel(True)(
        x1lo, x1hi, x2lo, x2hi, src4, dst4, edge_attr1, edge_attr2,
        z64, z16)
    halo, hahi, hblo, hbhi = _tc1(aXlo, aXhi, aE, wxlo0, wxhi0, we0, b0)
    bXlo, bXhi = _sc_kernel(False)(halo, hahi, hblo, hbhi, src4, dst4, z64)
    o1, o2 = _tc2(bXlo, bXhi, aE, wxlo1, wxhi1, we1, b1, wfcT, bfc2)
    return (o1, o2)
